# baseline (speedup 1.0000x reference)
import functools

import jax
import jax.numpy as jnp
from jax import lax
from jax.experimental import pallas as pl
from jax.experimental.pallas import tpu as pltpu
from jax.experimental.pallas import tpu_sc as plsc

MAXLEN = 200
VOCAB = 100000
EMBED = 64
BATCH = 4096

NC = 2
NS = 16
L = 16
NW = NC * NS
SEQ_PER_W = BATCH // NW
KSEQ = 4
NBLK = SEQ_PER_W // KSEQ
HALF = MAXLEN // 2
GPB = KSEQ * MAXLEN // HALF
ROWS = KSEQ * MAXLEN

_mesh = plsc.VectorSubcoreMesh(
    core_axis_name="c", subcore_axis_name="s", num_cores=NC, num_subcores=NS
)


@functools.partial(
    pl.kernel,
    out_type=jax.ShapeDtypeStruct((BATCH * MAXLEN, EMBED), jnp.float32),
    mesh=_mesh,
    scratch_types=[
        pltpu.VMEM((2, GPB, HALF), jnp.int32),
        pltpu.VMEM((MAXLEN, EMBED), jnp.float32),
        pltpu.VMEM((2, ROWS, EMBED), jnp.float32),
        pltpu.SemaphoreType.DMA((2,)),
        pltpu.SemaphoreType.DMA((2,)),
        pltpu.SemaphoreType.DMA((2,)),
    ],
    compiler_params=pltpu.CompilerParams(use_tc_tiling_on_sc=False),
)
def _tok_pos_embed(x_hbm, tok_hbm, pos_hbm, out_hbm, idx_v, pos_v, rows_v, isem, gsem, osem):
    wid = lax.axis_index("s") * NC + lax.axis_index("c")
    base = wid * SEQ_PER_W * 2

    def idx_copy(blk, buf):
        return pltpu.make_async_copy(
            x_hbm.at[pl.ds(base + blk * GPB, GPB)], idx_v.at[buf], isem.at[buf]
        )

    def gather(blk, buf, j):
        return pltpu.make_async_copy(
            tok_hbm.at[idx_v.at[buf].at[j]],
            rows_v.at[buf].at[pl.ds(j * HALF, HALF)],
            gsem.at[buf],
        )

    def out_copy(blk, buf):
        return pltpu.make_async_copy(
            rows_v.at[buf],
            out_hbm.at[pl.ds((wid * NBLK + blk) * ROWS, ROWS)],
            osem.at[buf],
        )

    pltpu.sync_copy(pos_hbm, pos_v)
    idx_copy(0, 0).start()
    idx_copy(0, 0).wait()
    for j in range(GPB):
        gather(0, 0, j).start()
    idx_copy(1, 1).start()

    @pl.loop(0, NBLK)
    def _blk(i):
        b = lax.rem(i, 2)
        nb = 1 - b

        for j in range(GPB):
            gather(i, b, j).wait()

        @pl.when(i + 2 < NBLK)
        def _():
            idx_copy(i + 2, b).start()

        @pl.when(i + 1 < NBLK)
        def _():
            idx_copy(i + 1, nb).wait()

            @pl.when(i >= 1)
            def _():
                out_copy(i - 1, nb).wait()

            for j in range(GPB):
                gather(i + 1, nb, j).start()

        @pl.loop(0, MAXLEN)
        def _row(r):
            for j in range(EMBED // L):
                sl = pl.ds(j * L, L)
                pv = pos_v[r, sl]
                for k in range(KSEQ):
                    rows_v[b, k * MAXLEN + r, sl] = rows_v[b, k * MAXLEN + r, sl] + pv

        out_copy(i, b).start()

    out_copy(NBLK - 2, lax.rem(NBLK - 2, 2)).wait()
    out_copy(NBLK - 1, lax.rem(NBLK - 1, 2)).wait()


def kernel(x, token_table, pos_table):
    x2 = x.astype(jnp.int32).reshape(2 * BATCH, HALF)
    out = _tok_pos_embed(x2, token_table, pos_table)
    return out.reshape(BATCH, MAXLEN, EMBED)


# speedup vs baseline: 1.4776x; 1.4776x over previous
import functools

import jax
import jax.numpy as jnp
from jax import lax
from jax.experimental import pallas as pl
from jax.experimental.pallas import tpu as pltpu
from jax.experimental.pallas import tpu_sc as plsc

MAXLEN = 200
VOCAB = 100000
EMBED = 64
BATCH = 4096

NC = 2
NS = 16
L = 16
NW = NC * NS
SEQ_PER_W = BATCH // NW
KSEQ = 4
NBLK = SEQ_PER_W // KSEQ
HALF = MAXLEN // 2
GPB = KSEQ * MAXLEN // HALF
ROWS = KSEQ * MAXLEN

_mesh = plsc.VectorSubcoreMesh(
    core_axis_name="c", subcore_axis_name="s", num_cores=NC, num_subcores=NS
)


@functools.partial(
    pl.kernel,
    out_type=jax.ShapeDtypeStruct((BATCH * MAXLEN, EMBED), jnp.float32),
    mesh=_mesh,
    scratch_types=[
        pltpu.VMEM((2, GPB, HALF), jnp.int32),
        pltpu.VMEM((MAXLEN, EMBED), jnp.float32),
        pltpu.VMEM((2, ROWS, EMBED), jnp.float32),
        pltpu.SemaphoreType.DMA((2,)),
        pltpu.SemaphoreType.DMA((2,)),
        pltpu.SemaphoreType.DMA((2,)),
    ],
    compiler_params=pltpu.CompilerParams(use_tc_tiling_on_sc=False),
)
def _tok_pos_embed(x_hbm, tok_hbm, pos_hbm, out_hbm, idx_v, pos_v, rows_v, isem, gsem, osem):
    wid = lax.axis_index("s") * NC + lax.axis_index("c")
    base = wid * SEQ_PER_W * 2

    def idx_copy(blk, buf):
        return pltpu.make_async_copy(
            x_hbm.at[pl.ds(base + blk * GPB, GPB)], idx_v.at[buf], isem.at[buf]
        )

    def gather(blk, buf, j):
        return pltpu.make_async_copy(
            tok_hbm.at[idx_v.at[buf].at[j]],
            rows_v.at[buf].at[pl.ds(j * HALF, HALF)],
            gsem.at[buf],
        )

    def out_copy(blk, buf):
        return pltpu.make_async_copy(
            rows_v.at[buf],
            out_hbm.at[pl.ds((wid * NBLK + blk) * ROWS, ROWS)],
            osem.at[buf],
        )

    pltpu.sync_copy(pos_hbm, pos_v)
    idx_copy(0, 0).start()
    idx_copy(0, 0).wait()
    for j in range(GPB):
        gather(0, 0, j).start()
    idx_copy(1, 1).start()

    def half(i, b):
        nb = 1 - b
        rows_b = rows_v.at[b]

        for j in range(GPB):
            gather(i, b, j).wait()

        @pl.when(i + 2 < NBLK)
        def _():
            idx_copy(i + 2, b).start()

        @pl.when(i + 1 < NBLK)
        def _():
            idx_copy(i + 1, nb).wait()

            @pl.when(i >= 1)
            def _():
                out_copy(i - 1, nb).wait()

            for j in range(GPB):
                gather(i + 1, nb, j).start()

        @pl.loop(0, MAXLEN)
        def _row(r):
            for j in range(EMBED // L):
                sl = pl.ds(j * L, L)
                pv = pos_v[r, sl]
                for k in range(KSEQ):
                    rows_b[k * MAXLEN + r, sl] = rows_b[k * MAXLEN + r, sl] + pv

        out_copy(i, b).start()

    @pl.loop(0, NBLK, step=2)
    def _blk(i):
        half(i, 0)
        half(i + 1, 1)

    out_copy(NBLK - 2, lax.rem(NBLK - 2, 2)).wait()
    out_copy(NBLK - 1, lax.rem(NBLK - 1, 2)).wait()


def kernel(x, token_table, pos_table):
    x2 = x.astype(jnp.int32).reshape(2 * BATCH, HALF)
    out = _tok_pos_embed(x2, token_table, pos_table)
    return out.reshape(BATCH, MAXLEN, EMBED)
